# Initial kernel scaffold; baseline (speedup 1.0000x reference)
import jax
import jax.numpy as jnp
from jax.experimental import pallas as pl


def kernel(rel_pos_bias, seq_len):
    raise NotImplementedError("write your pallas kernel here")


# speedup vs baseline: 9.7002x; 9.7002x over previous
import functools

import jax
import jax.numpy as jnp
from jax import lax
from jax.experimental import pallas as pl
from jax.experimental.pallas import tpu as pltpu
from jax.experimental.pallas import tpu_sc as plsc

HIDDEN = 16
MAX_LEN = 5000
SEQ = 2048
WIN = 2 * SEQ - 1
ROW = SEQ * HIDDEN
NC, NS = 2, 16
NW = NC * NS
ROWS_PER_W = SEQ // NW
RSUB = SEQ + ROWS_PER_W
DEPTH = 4


def _sc_expand(win_flat):
    mesh = plsc.VectorSubcoreMesh(core_axis_name="c", subcore_axis_name="s")

    @functools.partial(
        pl.kernel,
        mesh=mesh,
        out_type=jax.ShapeDtypeStruct((SEQ * ROW,), jnp.float32),
        scratch_types=[
            pltpu.VMEM((WIN * HIDDEN,), jnp.float32),
            pltpu.VMEM((RSUB * HIDDEN,), jnp.float32),
            pltpu.SemaphoreType.DMA,
            pltpu.SemaphoreType.DMA,
        ],
    )
    def k(win_hbm, out_hbm, fwd_v, rev_v, gsem, wsem):
        wid = lax.axis_index("s") * NC + lax.axis_index("c")

        pltpu.async_copy(win_hbm, fwd_v, gsem).wait()

        base = (RSUB - 2) + wid * ROWS_PER_W

        def rev8(q, _):
            for u in range(8):
                s = q * 8 + u
                m = jnp.maximum(base - s, 0)
                rev_v[pl.ds(s * HIDDEN, HIDDEN)] = fwd_v[pl.ds(m * HIDDEN, HIDDEN)]
            return _

        lax.fori_loop(0, RSUB // 8, rev8, None)

        row0 = wid * ROWS_PER_W

        def emit(r, _):
            cp = pltpu.make_async_copy(
                rev_v.at[pl.ds((ROWS_PER_W - 1 - r) * HIDDEN, ROW)],
                out_hbm.at[pl.ds((row0 + r) * ROW, ROW)],
                wsem,
            )
            cp.start()

            @pl.when(r >= DEPTH)
            def _drain():
                cp.wait()

            return _

        lax.fori_loop(0, ROWS_PER_W, emit, None)

        def drain(r, _):
            pltpu.make_async_copy(
                rev_v.at[pl.ds((ROWS_PER_W - 1 - r) * HIDDEN, ROW)],
                out_hbm.at[pl.ds((row0 + r) * ROW, ROW)],
                wsem,
            ).wait()
            return _

        lax.fori_loop(0, DEPTH, drain, None)

    return k(win_flat)


def kernel(rel_pos_bias, seq_len):
    start = jnp.asarray(seq_len, jnp.int32) - SEQ + (MAX_LEN - SEQ)
    win = lax.dynamic_slice_in_dim(rel_pos_bias, start, WIN, axis=0)
    out = _sc_expand(win.reshape(WIN * HIDDEN))
    return out.reshape(SEQ, SEQ, HIDDEN)


# speedup vs baseline: 40.1125x; 4.1352x over previous
import functools

import jax
import jax.numpy as jnp
from jax import lax
from jax.experimental import pallas as pl
from jax.experimental.pallas import tpu as pltpu
from jax.experimental.pallas import tpu_sc as plsc

HIDDEN = 16
MAX_LEN = 5000
SEQ = 2048
WIN = 2 * SEQ - 1
NC, NS = 2, 16
NW = NC * NS
ROWS_PER_W = SEQ // NW
RUSED = SEQ + 8 * (ROWS_PER_W - 1)
RLEN = 2560
DEPTH = 4


def _sc_expand(win_flat):
    mesh = plsc.VectorSubcoreMesh(core_axis_name="c", subcore_axis_name="s")

    @functools.partial(
        pl.kernel,
        mesh=mesh,
        out_type=jax.ShapeDtypeStruct((SEQ, HIDDEN, SEQ), jnp.float32),
        scratch_types=[
            pltpu.VMEM((WIN * HIDDEN,), jnp.float32),
            pltpu.VMEM((HIDDEN, RLEN), jnp.float32),
            pltpu.SemaphoreType.DMA,
            pltpu.SemaphoreType.DMA,
        ],
        compiler_params=pltpu.CompilerParams(
            use_tc_tiling_on_sc=False, needs_layout_passes=False
        ),
    )
    def k(win_hbm, out_hbm, fwd_v, rev_v, gsem, wsem):
        wid = lax.axis_index("s") * NC + lax.axis_index("c")
        b = wid // 8
        p = wid % 8

        pltpu.async_copy(win_hbm, fwd_v, gsem).wait()

        base = (RUSED - 1) + p + 512 * b
        lane = lax.iota(jnp.int32, 16)

        def rev4(q, _):
            for u in range(4):
                s = q * 4 + u
                v = fwd_v[pl.ds((base - s) * HIDDEN, HIDDEN)]
                plsc.store_scatter(rev_v, [lane, jnp.full((16,), s, jnp.int32)], v)
            return _

        lax.fori_loop(0, RUSED // 4, rev4, None)

        row0 = 512 * b + p

        def src(t):
            off = pl.multiple_of(8 * (ROWS_PER_W - 1 - t), 8)
            return rev_v.at[:, pl.ds(off, SEQ)]

        def emit(t, _):
            cp = pltpu.make_async_copy(src(t), out_hbm.at[row0 + 8 * t], wsem)
            cp.start()

            @pl.when(t >= DEPTH)
            def _drain():
                cp.wait()

            return _

        lax.fori_loop(0, ROWS_PER_W, emit, None)

        def drain(t, _):
            pltpu.make_async_copy(src(t), out_hbm.at[row0 + 8 * t], wsem).wait()
            return _

        lax.fori_loop(0, DEPTH, drain, None)

    return k(win_flat)


def kernel(rel_pos_bias, seq_len):
    start = jnp.asarray(seq_len, jnp.int32) - SEQ + (MAX_LEN - SEQ)
    win = lax.dynamic_slice_in_dim(rel_pos_bias, start, WIN, axis=0)
    out = _sc_expand(win.reshape(WIN * HIDDEN))
    return jnp.transpose(out, (0, 2, 1))


# speedup vs baseline: 96.8764x; 2.4151x over previous
import functools

import jax
import jax.numpy as jnp
from jax import lax
from jax.experimental import pallas as pl
from jax.experimental.pallas import tpu as pltpu
from jax.experimental.pallas import tpu_sc as plsc

HIDDEN = 16
MAX_LEN = 5000
SEQ = 2048
WIN = 2 * SEQ - 1
NC, NS = 2, 16
NW = NC * NS
NTASK = 8
NK = 16
FSTG = SEQ + 128 * (NK - 1) + NTASK
NBLK = (SEQ + 128 * (NK - 1)) // 128
TASKF = NBLK * 1024
SEG = (SEQ * HIDDEN) // 2


def _sc_expand(win_flat):
    mesh = plsc.VectorSubcoreMesh(core_axis_name="c", subcore_axis_name="s")

    @functools.partial(
        pl.kernel,
        mesh=mesh,
        out_type=jax.ShapeDtypeStruct((SEQ * SEQ * HIDDEN,), jnp.float32),
        scratch_types=[
            pltpu.VMEM((FSTG * HIDDEN,), jnp.float32),
            pltpu.VMEM((TASKF,), jnp.float32),
            pltpu.VMEM((TASKF,), jnp.float32),
            pltpu.SemaphoreType.DMA,
            pltpu.SemaphoreType.DMA,
        ],
        compiler_params=pltpu.CompilerParams(
            use_tc_tiling_on_sc=False, needs_layout_passes=False
        ),
    )
    def k(win_hbm, out_hbm, fwd_v, buf0_v, buf1_v, gsem, wsem):
        wid = lax.axis_index("s") * NC + lax.axis_index("c")
        a = wid // 2
        hb = wid % 2

        s = jnp.minimum(8 * a, WIN - FSTG)
        pltpu.async_copy(
            win_hbm.at[pl.ds(s * HIDDEN, FSTG * HIDDEN)], fwd_v, gsem
        ).wait()

        lane = lax.iota(jnp.int32, 16)
        neg16 = lane * (-16)

        def drain(n):
            for _ in range(n):
                pltpu.make_async_copy(
                    buf0_v.at[pl.ds(0, SEG)], out_hbm.at[pl.ds(0, SEG)], wsem
                ).wait()

        for t in range(NTASK):
            buf = buf0_v if t % 2 == 0 else buf1_v
            if t >= 2:
                drain(NK)
            p = 8 * a + t
            r0 = (3967 + p - s) * HIDDEN + 8 * hb

            def render(c, _):
                cb = r0 - c * (128 * HIDDEN)
                for h8 in range(8):
                    for jc in range(8):
                        idx = (cb - jc * (16 * HIDDEN) + h8) + neg16
                        v = plsc.load_gather(fwd_v, [idx])
                        base = c * 1024 + h8 * 128 + jc * 16
                        buf[pl.ds(base, 16)] = v
                return _

            lax.fori_loop(0, NBLK, render, None)

            def emit(kk, _):
                src = buf.at[pl.ds((NK - 1 - kk) * 1024, SEG)]
                dst = out_hbm.at[
                    pl.ds((p + 128 * kk) * (SEQ * HIDDEN) + hb * SEG, SEG)
                ]
                pltpu.make_async_copy(src, dst, wsem).start()
                return _

            lax.fori_loop(0, NK, emit, None)

        drain(2 * NK)

    return k(win_flat)


def kernel(rel_pos_bias, seq_len):
    start = jnp.asarray(seq_len, jnp.int32) - SEQ + (MAX_LEN - SEQ)
    win = lax.dynamic_slice_in_dim(rel_pos_bias, start, WIN, axis=0)
    flat = _sc_expand(win.reshape(WIN * HIDDEN))
    out5 = flat.reshape(SEQ, 2, 16, 8, 128)
    return out5.transpose(0, 2, 4, 1, 3).reshape(SEQ, SEQ, HIDDEN)
